# Initial kernel scaffold; baseline (speedup 1.0000x reference)
import jax
import jax.numpy as jnp
from jax.experimental import pallas as pl


def kernel(E_t, batch):
    raise NotImplementedError("write your pallas kernel here")


# speedup vs baseline: 1.8265x; 1.8265x over previous
import functools

import jax
import jax.numpy as jnp
from jax import lax
from jax.experimental import pallas as pl
from jax.experimental.pallas import tpu as pltpu
from jax.experimental.pallas import tpu_sc as plsc

_NC = 2
_NS = 16
_NW = _NC * _NS
_LANES = 16
_CHUNK = 256


def _seg_sum_call(n_atoms, d, n_mol):
    m_per_w = n_mol // _NW
    n_col_grp = d // _LANES

    mesh = plsc.VectorSubcoreMesh(
        core_axis_name="c", subcore_axis_name="s",
        num_cores=_NC, num_subcores=_NS)

    @functools.partial(
        pl.kernel,
        out_type=jax.ShapeDtypeStruct((n_mol, d), jnp.float32),
        mesh=mesh,
        scratch_types=[
            pltpu.VMEM((48,), jnp.int32),
            pltpu.VMEM((_CHUNK + 16,), jnp.int32),
            pltpu.VMEM((_CHUNK, d), jnp.float32),
            pltpu.VMEM((m_per_w, d), jnp.float32),
        ],
    )
    def seg_sum(e_hbm, batch_hbm, bounds_hbm, out_hbm,
                bounds_v, ids_v, rows_v, acc_v):
        wid = lax.axis_index("c") * _NS + lax.axis_index("s")
        pltpu.sync_copy(bounds_hbm, bounds_v)
        bvec = bounds_v[pl.ds(wid, 16)]
        a0 = bvec[0]
        a1 = bvec[1]
        m0 = wid * m_per_w

        zeros = jnp.zeros((_LANES,), jnp.float32)

        def zero_body(i, carry):
            for c in range(n_col_grp):
                acc_v[i, pl.ds(c * _LANES, _LANES)] = zeros
            return carry

        lax.fori_loop(0, m_per_w, zero_body, 0)

        base = a0 & ~7
        n_chunks = (a1 - base + _CHUNK - 1) // _CHUNK

        def chunk_body(g, carry):
            raw_start = base + g * _CHUNK
            start = pl.multiple_of(jnp.minimum(raw_start, n_atoms - _CHUNK), 8)
            pltpu.sync_copy(batch_hbm.at[pl.ds(start, _CHUNK)],
                            ids_v.at[pl.ds(0, _CHUNK)])
            pltpu.sync_copy(e_hbm.at[pl.ds(start, _CHUNK), :], rows_v)
            lo = jnp.maximum(a0, raw_start) - start
            hi = jnp.minimum(a1, raw_start + _CHUNK) - start

            def row_body(r, inner):
                seg = ids_v[pl.ds(r, 16)][0] - m0
                for c in range(n_col_grp):
                    sl = pl.ds(c * _LANES, _LANES)
                    plsc.addupdate(acc_v.at[seg, sl], rows_v[r, sl])
                return inner

            lax.fori_loop(lo, hi, row_body, 0)
            return carry

        lax.fori_loop(0, n_chunks, chunk_body, 0)
        pltpu.sync_copy(acc_v, out_hbm.at[pl.ds(m0, m_per_w), :])

    return seg_sum


def kernel(E_t, batch):
    n_atoms, d = E_t.shape
    n_mol = 4096
    m_per_w = n_mol // _NW
    mol_starts = jnp.arange(_NW + 1, dtype=jnp.int32) * m_per_w
    bounds = jnp.searchsorted(batch, mol_starts, side="left").astype(jnp.int32)
    bounds = jnp.concatenate([bounds, jnp.zeros((15,), jnp.int32)])
    return _seg_sum_call(n_atoms, d, n_mol)(E_t, batch, bounds)


# speedup vs baseline: 2.3530x; 1.2883x over previous
import functools

import jax
import jax.numpy as jnp
from jax import lax
from jax.experimental import pallas as pl
from jax.experimental.pallas import tpu as pltpu
from jax.experimental.pallas import tpu_sc as plsc

_NC = 2
_NS = 16
_NW = _NC * _NS
_LANES = 16
_CHUNK = 256


def _seg_sum_call(n_atoms, d, n_mol):
    m_per_w = n_mol // _NW
    n_col_grp = d // _LANES

    mesh = plsc.VectorSubcoreMesh(
        core_axis_name="c", subcore_axis_name="s",
        num_cores=_NC, num_subcores=_NS)

    @functools.partial(
        pl.kernel,
        out_type=jax.ShapeDtypeStruct((n_mol, d), jnp.float32),
        mesh=mesh,
        scratch_types=[
            pltpu.VMEM((48,), jnp.int32),
            pltpu.VMEM((2 * _CHUNK,), jnp.int32),
            pltpu.VMEM((2, _CHUNK, d), jnp.float32),
            pltpu.VMEM((m_per_w, d), jnp.float32),
            pltpu.SemaphoreType.DMA((2,)),
        ],
    )
    def seg_sum(e_hbm, batch_hbm, bounds_hbm, out_hbm,
                bounds_v, ids_v, rows_v, acc_v, sems):
        wid = lax.axis_index("c") * _NS + lax.axis_index("s")
        pltpu.sync_copy(bounds_hbm, bounds_v)
        bvec = bounds_v[pl.ds(wid, 16)]
        a0 = bvec[0]
        a1 = bvec[1]
        m0 = wid * m_per_w

        zeros = jnp.zeros((_LANES,), jnp.float32)

        def zero_body(i, carry):
            for c in range(n_col_grp):
                acc_v[i, pl.ds(c * _LANES, _LANES)] = zeros
            return carry

        lax.fori_loop(0, m_per_w, zero_body, 0)

        base = a0 & ~7
        n_chunks = (a1 - base + _CHUNK - 1) // _CHUNK

        def chunk_refs(g):
            slot = g % 2
            raw_start = base + g * _CHUNK
            start = pl.multiple_of(
                jnp.minimum(raw_start, n_atoms - _CHUNK), 8)
            return (slot, raw_start, start,
                    batch_hbm.at[pl.ds(start, _CHUNK)],
                    ids_v.at[pl.ds(pl.multiple_of(slot * _CHUNK, 128),
                                   _CHUNK)],
                    e_hbm.at[pl.ds(start, _CHUNK), :],
                    rows_v.at[slot])

        def start_chunk(g):
            slot, _, _, ids_src, ids_dst, row_src, row_dst = chunk_refs(g)
            pltpu.async_copy(ids_src, ids_dst, sems.at[slot])
            pltpu.async_copy(row_src, row_dst, sems.at[slot])

        @pl.when(n_chunks > 0)
        def _():
            start_chunk(0)

        def chunk_body(g, carry):
            @pl.when(g + 1 < n_chunks)
            def _():
                start_chunk(g + 1)

            slot, raw_start, start, ids_src, ids_dst, row_src, row_dst = (
                chunk_refs(g))
            pltpu.make_async_copy(ids_src, ids_dst, sems.at[slot]).wait()
            pltpu.make_async_copy(row_src, row_dst, sems.at[slot]).wait()

            lo = jnp.maximum(a0, raw_start) - start
            hi = jnp.minimum(a1, raw_start + _CHUNK) - start

            def block_body(b, inner):
                off = pl.multiple_of(b * _LANES, _LANES)
                ivec = ids_v[pl.ds(
                    pl.multiple_of(slot * _CHUNK + off, _LANES), _LANES)]
                for j in range(_LANES):
                    r = off + j
                    ok = (r >= lo) & (r < hi)
                    seg = jnp.clip(ivec[j] - m0, 0, m_per_w - 1)
                    for c in range(n_col_grp):
                        sl = pl.ds(c * _LANES, _LANES)
                        val = jnp.where(ok, rows_v[slot, r, sl], zeros)
                        plsc.addupdate(acc_v.at[seg, sl], val)
                return inner

            lax.fori_loop(lo // _LANES, (hi + _LANES - 1) // _LANES,
                          block_body, 0)
            return carry

        lax.fori_loop(0, n_chunks, chunk_body, 0)
        pltpu.sync_copy(acc_v, out_hbm.at[pl.ds(m0, m_per_w), :])

    return seg_sum


def kernel(E_t, batch):
    n_atoms, d = E_t.shape
    n_mol = 4096
    m_per_w = n_mol // _NW
    mol_starts = jnp.arange(_NW + 1, dtype=jnp.int32) * m_per_w
    bounds = jnp.searchsorted(batch, mol_starts, side="left").astype(jnp.int32)
    bounds = jnp.concatenate([bounds, jnp.zeros((15,), jnp.int32)])
    return _seg_sum_call(n_atoms, d, n_mol)(E_t, batch, bounds)


# speedup vs baseline: 3.0935x; 1.3147x over previous
import functools

import jax
import jax.numpy as jnp
from jax import lax
from jax.experimental import pallas as pl
from jax.experimental.pallas import tpu as pltpu
from jax.experimental.pallas import tpu_sc as plsc

_NC = 2
_NS = 16
_NW = _NC * _NS
_LANES = 16
_CHUNK = 256


def _seg_sum_call(n_atoms, d, n_mol):
    m_per_w = n_mol // _NW
    n_col_grp = d // _LANES

    mesh = plsc.VectorSubcoreMesh(
        core_axis_name="c", subcore_axis_name="s",
        num_cores=_NC, num_subcores=_NS)

    @functools.partial(
        pl.kernel,
        out_type=jax.ShapeDtypeStruct((n_mol, d), jnp.float32),
        mesh=mesh,
        scratch_types=[
            pltpu.VMEM((48,), jnp.int32),
            pltpu.VMEM((2 * _CHUNK,), jnp.int32),
            pltpu.VMEM((2, _CHUNK, d), jnp.float32),
            pltpu.VMEM((m_per_w, d), jnp.float32),
            pltpu.SemaphoreType.DMA((2,)),
        ],
    )
    def seg_sum(e_hbm, batch_hbm, bounds_hbm, out_hbm,
                bounds_v, ids_v, rows_v, acc_v, sems):
        wid = lax.axis_index("c") * _NS + lax.axis_index("s")
        pltpu.sync_copy(bounds_hbm, bounds_v)
        bvec = bounds_v[pl.ds(wid, 16)]
        a0 = bvec[0]
        a1 = bvec[1]
        m0 = wid * m_per_w

        zeros = jnp.zeros((_LANES,), jnp.float32)

        def zero_body(i, carry):
            for c in range(n_col_grp):
                acc_v[i, pl.ds(c * _LANES, _LANES)] = zeros
            return carry

        lax.fori_loop(0, m_per_w, zero_body, 0)

        base = a0 & ~7
        n_chunks = (a1 - base + _CHUNK - 1) // _CHUNK

        def chunk_refs(g):
            slot = g % 2
            raw_start = base + g * _CHUNK
            start = pl.multiple_of(
                jnp.minimum(raw_start, n_atoms - _CHUNK), 8)
            return (slot, raw_start, start,
                    batch_hbm.at[pl.ds(start, _CHUNK)],
                    ids_v.at[pl.ds(pl.multiple_of(slot * _CHUNK, 128),
                                   _CHUNK)],
                    e_hbm.at[pl.ds(start, _CHUNK), :],
                    rows_v.at[slot])

        def start_chunk(g):
            slot, _, _, ids_src, ids_dst, row_src, row_dst = chunk_refs(g)
            pltpu.async_copy(ids_src, ids_dst, sems.at[slot])
            pltpu.async_copy(row_src, row_dst, sems.at[slot])

        @pl.when(n_chunks > 0)
        def _():
            start_chunk(0)

        def chunk_body(g, carry):
            @pl.when(g + 1 < n_chunks)
            def _():
                start_chunk(g + 1)

            slot, raw_start, start, ids_src, ids_dst, row_src, row_dst = (
                chunk_refs(g))
            pltpu.make_async_copy(ids_src, ids_dst, sems.at[slot]).wait()
            pltpu.make_async_copy(row_src, row_dst, sems.at[slot]).wait()

            lo = jnp.maximum(a0, raw_start) - start
            hi = jnp.minimum(a1, raw_start + _CHUNK) - start

            @plsc.parallel_loop(lo // _LANES, (hi + _LANES - 1) // _LANES, 1,
                                unroll=2)
            def _(b):
                off = pl.multiple_of(b * _LANES, _LANES)
                ivec = ids_v[pl.ds(
                    pl.multiple_of(slot * _CHUNK + off, _LANES), _LANES)]
                for j in range(_LANES):
                    r = off + j
                    ok = (r >= lo) & (r < hi)
                    seg = jnp.clip(ivec[j] - m0, 0, m_per_w - 1)
                    for c in range(n_col_grp):
                        sl = pl.ds(c * _LANES, _LANES)
                        val = jnp.where(ok, rows_v[slot, r, sl], zeros)
                        plsc.addupdate(acc_v.at[seg, sl], val)

            return carry

        lax.fori_loop(0, n_chunks, chunk_body, 0)
        pltpu.sync_copy(acc_v, out_hbm.at[pl.ds(m0, m_per_w), :])

    return seg_sum


def kernel(E_t, batch):
    n_atoms, d = E_t.shape
    n_mol = 4096
    m_per_w = n_mol // _NW
    mol_starts = jnp.arange(_NW + 1, dtype=jnp.int32) * m_per_w
    bounds = jnp.searchsorted(batch, mol_starts, side="left").astype(jnp.int32)
    bounds = jnp.concatenate([bounds, jnp.zeros((15,), jnp.int32)])
    return _seg_sum_call(n_atoms, d, n_mol)(E_t, batch, bounds)
